# initial kernel scaffold (unmeasured)
import jax
import jax.numpy as jnp
from jax import lax
from jax.experimental import pallas as pl
from jax.experimental.pallas import tpu as pltpu

T = 2048
D = 4096
V_SHARD = 8192
VT = 512
NT = V_SHARD // VT


def kernel(x, W, labels):
    labels_col = labels.reshape(T, 1)

    def body(x_ref, w_ref, lab_ref, out_ref,
             xbf_ref, s_ref, ll_ref, send_ref, recv_ref,
             send_sem, recv_sem):
        i = pl.program_id(0)
        my_y = lax.axis_index("y")

        @pl.when(i == 0)
        def _init():
            xbf_ref[...] = x_ref[...].astype(jnp.bfloat16)
            s_ref[...] = jnp.zeros_like(s_ref)
            ll_ref[...] = jnp.zeros_like(ll_ref)

        wbf = w_ref[...].astype(jnp.bfloat16)
        logits = jnp.dot(xbf_ref[...], wbf,
                         preferred_element_type=jnp.float32)

        s_ref[...] += jnp.sum(jnp.exp(logits), axis=1, keepdims=True)

        v0 = my_y * V_SHARD + i * VT
        vocab_ids = lax.broadcasted_iota(jnp.int32, (T, VT), 1) + v0
        mask = vocab_ids == lab_ref[...]
        ll_ref[...] += jnp.sum(jnp.where(mask, logits, 0.0),
                               axis=1, keepdims=True)

        @pl.when(i == NT - 1)
        def _finish():
            my_x = lax.axis_index("x")
            my_z = lax.axis_index("z")
            send_ref[0:1, :] = s_ref[...].reshape(1, T)
            send_ref[1:2, :] = ll_ref[...].reshape(1, T)
            rdma = pltpu.make_async_remote_copy(
                src_ref=send_ref,
                dst_ref=recv_ref,
                send_sem=send_sem,
                recv_sem=recv_sem,
                device_id=(my_x, 1 - my_y, my_z),
                device_id_type=pl.DeviceIdType.MESH,
            )
            rdma.start()
            rdma.wait()
            s_tot = send_ref[0:1, :] + recv_ref[0:1, :]
            ll_tot = send_ref[1:2, :] + recv_ref[1:2, :]
            out_ref[...] = jnp.log(s_tot) - ll_tot

    out = pl.pallas_call(
        body,
        grid=(NT,),
        in_specs=[
            pl.BlockSpec((T, D), lambda i: (0, 0)),
            pl.BlockSpec((D, VT), lambda i: (0, i)),
            pl.BlockSpec((T, 1), lambda i: (0, 0)),
        ],
        out_specs=pl.BlockSpec((1, T), lambda i: (0, 0)),
        out_shape=jax.ShapeDtypeStruct((1, T), jnp.float32),
        scratch_shapes=[
            pltpu.VMEM((T, D), jnp.bfloat16),
            pltpu.VMEM((T, 1), jnp.float32),
            pltpu.VMEM((T, 1), jnp.float32),
            pltpu.VMEM((2, T), jnp.float32),
            pltpu.VMEM((2, T), jnp.float32),
            pltpu.SemaphoreType.DMA,
            pltpu.SemaphoreType.DMA,
        ],
        compiler_params=pltpu.CompilerParams(collective_id=0),
    )(x, W, labels_col)
    return out.reshape(T)


# baseline (device time: 188394 ns/iter reference)
import jax
import jax.numpy as jnp
from jax import lax
from jax.experimental import pallas as pl
from jax.experimental.pallas import tpu as pltpu

T = 2048
D = 4096
V_SHARD = 8192
VT = 512
NT = V_SHARD // VT


def kernel(x, W, labels):
    labels_col = labels.reshape(T, 1)
    xbf = x.astype(jnp.bfloat16)

    def body(x_ref, w_ref, lab_ref, out_ref,
             s_ref, ll_ref, send_ref, recv_ref,
             send_sem, recv_sem):
        i = pl.program_id(0)
        my_y = lax.axis_index("y")

        @pl.when(i == 0)
        def _init():
            s_ref[...] = jnp.zeros_like(s_ref)
            ll_ref[...] = jnp.zeros_like(ll_ref)

        wbf = w_ref[...].astype(jnp.bfloat16)
        logits = jnp.dot(x_ref[...], wbf,
                         preferred_element_type=jnp.float32)

        s_ref[...] += jnp.sum(jnp.exp(logits), axis=1, keepdims=True)

        v0 = my_y * V_SHARD + i * VT
        vocab_ids = lax.broadcasted_iota(jnp.int32, (T, VT), 1) + v0
        mask = vocab_ids == lab_ref[...]
        ll_ref[...] += jnp.sum(jnp.where(mask, logits, 0.0),
                               axis=1, keepdims=True)

        @pl.when(i == NT - 1)
        def _finish():
            my_x = lax.axis_index("x")
            my_z = lax.axis_index("z")
            send_ref[0:1, :] = s_ref[...].reshape(1, T)
            send_ref[1:2, :] = ll_ref[...].reshape(1, T)
            rdma = pltpu.make_async_remote_copy(
                src_ref=send_ref,
                dst_ref=recv_ref,
                send_sem=send_sem,
                recv_sem=recv_sem,
                device_id=(my_x, 1 - my_y, my_z),
                device_id_type=pl.DeviceIdType.MESH,
            )
            rdma.start()
            rdma.wait()
            s_tot = send_ref[0:1, :] + recv_ref[0:1, :]
            ll_tot = send_ref[1:2, :] + recv_ref[1:2, :]
            out_ref[...] = jnp.log(s_tot) - ll_tot

    out = pl.pallas_call(
        body,
        grid=(NT,),
        in_specs=[
            pl.BlockSpec((T, D), lambda i: (0, 0)),
            pl.BlockSpec((D, VT), lambda i: (0, i)),
            pl.BlockSpec((T, 1), lambda i: (0, 0)),
        ],
        out_specs=pl.BlockSpec((1, T), lambda i: (0, 0)),
        out_shape=jax.ShapeDtypeStruct((1, T), jnp.float32),
        scratch_shapes=[
            pltpu.VMEM((T, 1), jnp.float32),
            pltpu.VMEM((T, 1), jnp.float32),
            pltpu.VMEM((2, T), jnp.float32),
            pltpu.VMEM((2, T), jnp.float32),
            pltpu.SemaphoreType.DMA,
            pltpu.SemaphoreType.DMA,
        ],
    )(xbf, W, labels_col)
    return out.reshape(T)


# device time: 72136 ns/iter; 2.6117x vs baseline; 2.6117x over previous
import jax
import jax.numpy as jnp
from jax import lax
from jax.experimental import pallas as pl
from jax.experimental.pallas import tpu as pltpu

T = 2048
D = 4096
V_SHARD = 8192
V_SUB = 2048
VT = 512
NT = V_SUB // VT

N_STAGES = 3


def kernel(x, W, labels):
    labels_col = labels.reshape(T, 1)
    xbf = x.astype(jnp.bfloat16)
    q = 2 * lax.axis_index("x") + lax.axis_index("z")
    q_arr = jnp.asarray(q, jnp.int32).reshape(1)

    def body(q_ref, x_ref, w_ref, lab_ref, out_ref,
             s_ref, ll_ref, send_ref, recv_ref,
             send_sem, recv_sems):
        i = pl.program_id(0)
        my_x = lax.axis_index("x")
        my_y = lax.axis_index("y")
        my_z = lax.axis_index("z")

        @pl.when(i == 0)
        def _init():
            s_ref[...] = jnp.zeros_like(s_ref)
            ll_ref[...] = jnp.zeros_like(ll_ref)

        wbf = w_ref[...].astype(jnp.bfloat16)
        logits = jnp.dot(x_ref[...], wbf,
                         preferred_element_type=jnp.float32)

        s_ref[...] += jnp.sum(jnp.exp(logits), axis=1, keepdims=True)

        v0 = my_y * V_SHARD + q_ref[0] * V_SUB + i * VT
        vocab_ids = lax.broadcasted_iota(jnp.int32, (T, VT), 1) + v0
        mask = vocab_ids == lab_ref[...]
        ll_ref[...] += jnp.sum(jnp.where(mask, logits, 0.0),
                               axis=1, keepdims=True)

        @pl.when(i == NT - 1)
        def _allreduce():
            acc = jnp.concatenate(
                [s_ref[...].reshape(1, T), ll_ref[...].reshape(1, T)],
                axis=0)
            partners = [
                (1 - my_x, my_y, my_z),
                (my_x, 1 - my_y, my_z),
                (my_x, my_y, 1 - my_z),
            ]
            for st in range(N_STAGES):
                send_ref[...] = acc
                rdma = pltpu.make_async_remote_copy(
                    src_ref=send_ref,
                    dst_ref=recv_ref.at[st],
                    send_sem=send_sem,
                    recv_sem=recv_sems.at[st],
                    device_id=partners[st],
                    device_id_type=pl.DeviceIdType.MESH,
                )
                rdma.start()
                rdma.wait()
                acc = acc + recv_ref[st]
            out_ref[...] = jnp.log(acc[0:1, :]) - acc[1:2, :]

    grid_spec = pltpu.PrefetchScalarGridSpec(
        num_scalar_prefetch=1,
        grid=(NT,),
        in_specs=[
            pl.BlockSpec((T, D), lambda i, q: (0, 0)),
            pl.BlockSpec((D, VT), lambda i, q: (0, q[0] * NT + i)),
            pl.BlockSpec((T, 1), lambda i, q: (0, 0)),
        ],
        out_specs=pl.BlockSpec((1, T), lambda i, q: (0, 0)),
        scratch_shapes=[
            pltpu.VMEM((T, 1), jnp.float32),
            pltpu.VMEM((T, 1), jnp.float32),
            pltpu.VMEM((2, T), jnp.float32),
            pltpu.VMEM((N_STAGES, 2, T), jnp.float32),
            pltpu.SemaphoreType.DMA,
            pltpu.SemaphoreType.DMA((N_STAGES,)),
        ],
    )

    out = pl.pallas_call(
        body,
        grid_spec=grid_spec,
        out_shape=jax.ShapeDtypeStruct((1, T), jnp.float32),
    )(q_arr, xbf, W, labels_col)
    return out.reshape(T)
